# Initial kernel scaffold; baseline (speedup 1.0000x reference)
import jax
import jax.numpy as jnp
from jax.experimental import pallas as pl


def kernel(x, edge_index, batch, params):
    raise NotImplementedError("write your pallas kernel here")


# speedup vs baseline: 1.8038x; 1.8038x over previous
import functools

import numpy as np
import jax
import jax.numpy as jnp
from jax import lax
from jax.experimental import pallas as pl
from jax.experimental.pallas import tpu as pltpu
from jax.experimental.pallas import tpu_sc as plsc

_N = 10000
_E = 320000
_D = 128
_H = 256
_LAT = 64
_B = 512
_NMAX = 128
_NE = _NMAX * (_NMAX - 1) // 2
_LG = 8192
_EPS = 1e-5

_NTILE = 16
_NPAD = 10240

_WIN_D128 = [10080] * 11 + [9840] * 4 + [9760]
_WIN_D256 = [10080] * 5 + [9968] * 10 + [9920]
_CH128 = 80
_CH256 = 112


def _win_tables(sizes):
    starts = np.concatenate([[0], np.cumsum(sizes)[:-1]]).astype(np.int32)
    return starts


_ST128 = _win_tables(_WIN_D128)
_ST256 = _win_tables(_WIN_D256)
_BOUNDS = np.cumsum(np.array(_WIN_D128 + _WIN_D128, np.int64))[:-1].astype(np.int32)
_BOUNDS256 = np.cumsum(np.array(_WIN_D256 + _WIN_D256, np.int64))[:-1].astype(np.int32)


def _edge_window_ids(bounds):
    w = np.zeros((_E,), np.int32)
    w[bounds] = 1
    return np.cumsum(w).astype(np.int32)


_WID128 = _edge_window_ids(_BOUNDS)
_WID256 = _edge_window_ids(_BOUNDS256)


def _build_triu_map() -> np.ndarray:
    m = np.full((_NMAX, _NMAX), _LG - 1, np.int32)
    iu = np.triu_indices(_NMAX, 1)
    ks = np.arange(_NE, dtype=np.int32)
    m[iu] = ks
    m[(iu[1], iu[0])] = ks
    return m.reshape(-1)


_TRIU_MAP = _build_triu_map()


_TRASH_RMW = _N + 40


def _window_start_flags(sizes):
    flags = np.zeros((_E,), bool)
    pos = 0
    for w in sizes * 2:
        flags[pos] = True
        pos += w
    return flags


_WSF128 = _window_start_flags(_WIN_D128)
_WSF256 = _window_start_flags(_WIN_D256)


def _prep_edges(src, dst, bounds, wid, wsf):
    order = jnp.argsort(dst)
    ss = src[order]
    ds = dst[order]
    bv = jnp.asarray(bounds)
    prevlast = ds[bv - 1]
    widv = jnp.asarray(wid)
    pl_e = jnp.concatenate([ds[:1], prevlast])[widv]
    cond = (widv > 0) & (ds == pl_e)
    ds2 = jnp.where(cond, _N + widv, ds)
    merge = jnp.concatenate(
        [jnp.full((1,), _N, jnp.int32), ds[bv]])
    wsfv = jnp.asarray(wsf)
    same_prev = jnp.concatenate(
        [jnp.zeros((1,), bool), ds2[1:] == ds2[:-1]]) & ~wsfv
    carry = same_prev.astype(jnp.float32)
    carry16 = jnp.broadcast_to(carry[:, None], (_E, 16)).reshape(-1)
    run_cont = jnp.concatenate(
        [same_prev[1:], jnp.zeros((1,), bool)])
    sdst = jnp.where(run_cont, _TRASH_RMW, ds2)
    return ss, sdst, carry16, merge


def _acc_rows(S):
    return ((S + 32 + 127) // 128) * 128


@functools.cache
def _seg_sum_det_kernel(S: int, Dh: int, edge_split: bool):
    acc_r = _acc_rows(S)
    zrows = acc_r // _NTILE
    orows = (S // _NTILE // 8) * 8
    orem = S - _NTILE * orows
    mesh = plsc.VectorSubcoreMesh(core_axis_name="c", subcore_axis_name="s")
    ch = _CH128 if edge_split else _CH256

    @functools.partial(
        pl.kernel, mesh=mesh,
        out_type=(jax.ShapeDtypeStruct((S, Dh), jnp.float32),
                  jax.ShapeDtypeStruct((S, Dh), jnp.float32)),
        scratch_types=[
            pltpu.VMEM((ch,), jnp.int32),
            pltpu.VMEM((ch,), jnp.int32),
            pltpu.VMEM((ch, Dh), jnp.float32),
            pltpu.VMEM((ch * 16,), jnp.float32),
            pltpu.VMEM((64,), jnp.int32),
            pltpu.VMEM((64,), jnp.int32),
            pltpu.VMEM((64, Dh), jnp.float32),
            pltpu.VMEM((64 * 16,), jnp.float32),
            pltpu.VMEM((32,), jnp.int32),
            pltpu.VMEM((32, Dh), jnp.float32),
            pltpu.VMEM_SHARED((acc_r, Dh), jnp.float32),
            pltpu.SemaphoreType.DMA,
        ],
    )
    def k(src_hbm, dst_hbm, carry_hbm, h0_hbm, h1_hbm, z_hbm, merge_hbm,
          agg0_hbm, agg1_hbm,
          idxs_v, idxd_v, rows_v, carry_v, idxs64_v, idxd64_v, rows64_v,
          carry64_v, midx_v, bbuf_v, acc_sh, sem):
        c = lax.axis_index("c")
        s = lax.axis_index("s")
        nvec = Dh // 16
        zvec = tuple(jnp.zeros((16,), jnp.float32) for _ in range(nvec))

        def chunk(off, idxs, idxd, rows, carryv, h_hbm, prev):
            csz = idxs.shape[0]
            pltpu.sync_copy(src_hbm.at[pl.ds(off, csz)], idxs)
            pltpu.sync_copy(dst_hbm.at[pl.ds(off, csz)], idxd)
            pltpu.sync_copy(carry_hbm.at[pl.ds(off * 16, csz * 16)], carryv)
            pltpu.async_copy(h_hbm.at[idxs], rows, sem).wait()

            def sbody(i, pv):
                sv = carryv[pl.ds(i * 16, 16)]
                new = []
                for j in range(nvec):
                    cur = rows[i, pl.ds(j * 16, 16)]
                    nv = cur + pv[j] * sv
                    rows[i, pl.ds(j * 16, 16)] = nv
                    new.append(nv)
                return tuple(new)

            prev = lax.fori_loop(0, csz, sbody, prev)
            pltpu.sync_copy(rows, acc_sh.at[idxd], add=True)
            return prev

        def run(h_hbm, out_hbm):
            pltpu.sync_copy(z_hbm.at[pl.ds(0, zrows)],
                            acc_sh.at[pl.ds(s * zrows, zrows)])
            plsc.subcore_barrier()

            if edge_split:
                start = c * 160000 + jnp.where(
                    s < 11, 10080 * s,
                    jnp.where(s < 15, 110880 + 9840 * (s - 11), 150240))
                nch = jnp.where(s < 11, 126, jnp.where(s < 15, 123, 122))

                def body(j, pv):
                    return chunk(start + j * ch, idxs_v, idxd_v, rows_v,
                                 carry_v, h_hbm, pv)

                lax.fori_loop(0, nch, body, zvec)
            else:
                wstart = jnp.where(
                    s < 5, 10080 * s,
                    jnp.where(s < 15, 50400 + 9968 * (s - 5), 150080))
                nch = jnp.where(s < 5, 90, jnp.where(s < 15, 89, 88))
                for half in (0, 1):
                    start = half * 160000 + wstart

                    def body(j, pv, start=start):
                        return chunk(start + j * ch, idxs_v, idxd_v,
                                     rows_v, carry_v, h_hbm, pv)

                    pv = lax.fori_loop(0, nch, body, zvec)

                    @pl.when(s == 15)
                    def _():
                        chunk(start + 88 * ch, idxs64_v, idxd64_v,
                              rows64_v, carry64_v, h_hbm, pv)

            plsc.subcore_barrier()

            @pl.when(s == 0)
            def _():
                pltpu.sync_copy(merge_hbm, midx_v)
                pltpu.sync_copy(acc_sh.at[pl.ds(S, 32)], bbuf_v)
                pltpu.sync_copy(bbuf_v, acc_sh.at[midx_v], add=True)

            plsc.subcore_barrier()
            pltpu.sync_copy(acc_sh.at[pl.ds(s * orows, orows)],
                            out_hbm.at[pl.ds(s * orows, orows)])
            if orem:
                @pl.when(s == _NTILE - 1)
                def _():
                    pltpu.sync_copy(
                        acc_sh.at[pl.ds(_NTILE * orows, orem)],
                        out_hbm.at[pl.ds(_NTILE * orows, orem)])

        @pl.when(c == 0)
        def _():
            run(h0_hbm, agg0_hbm)

        @pl.when(c == 1)
        def _():
            run(h1_hbm, agg1_hbm)

    return k


def _seg_sum_conv(h, ss, sdst, carry, merge):
    rows, D = h.shape
    if D == _D:
        zeros = jnp.zeros((_acc_rows(_N) // _NTILE, _D), jnp.float32)
        a0, a1 = _seg_sum_det_kernel(_N, _D, True)(
            ss, sdst, carry, h, h, zeros, merge)
        return a0, a1
    Dh = D // 2
    zeros = jnp.zeros((_acc_rows(_N) // _NTILE, Dh), jnp.float32)
    a0, a1 = _seg_sum_det_kernel(_N, Dh, False)(
        ss, sdst, carry, h[:, :Dh], h[:, Dh:], zeros, merge)
    return (jnp.concatenate([a0, a1], axis=1),)


@functools.cache
def _seg_sum_pool_kernel(S: int, Dh: int, Epad: int):
    acc_r = _acc_rows(S)
    etile = Epad // _NTILE
    nchunk = etile // 128
    zrows = acc_r // _NTILE
    orows = (S // _NTILE // 8) * 8
    mesh = plsc.VectorSubcoreMesh(core_axis_name="c", subcore_axis_name="s")

    @functools.partial(
        pl.kernel, mesh=mesh,
        out_type=(jax.ShapeDtypeStruct((S, Dh), jnp.float32),
                  jax.ShapeDtypeStruct((S, Dh), jnp.float32)),
        scratch_types=[
            pltpu.VMEM((128,), jnp.int32),
            pltpu.VMEM((128,), jnp.int32),
            pltpu.VMEM((128, Dh), jnp.float32),
            pltpu.VMEM_SHARED((acc_r, Dh), jnp.float32),
            pltpu.SemaphoreType.DMA,
        ],
    )
    def k(src_hbm, dst_hbm, h0_hbm, h1_hbm, z_hbm, agg0_hbm, agg1_hbm,
          idxs_v, idxd_v, rows_v, acc_sh, sem):
        c = lax.axis_index("c")
        s = lax.axis_index("s")

        def run(h_hbm, out_hbm):
            pltpu.sync_copy(z_hbm.at[pl.ds(0, zrows)],
                            acc_sh.at[pl.ds(s * zrows, zrows)])
            plsc.subcore_barrier()
            base = s * etile

            def body(j, carry):
                off = base + j * 128
                pltpu.sync_copy(src_hbm.at[pl.ds(off, 128)], idxs_v)
                pltpu.sync_copy(dst_hbm.at[pl.ds(off, 128)], idxd_v)
                pltpu.async_copy(h_hbm.at[idxs_v], rows_v, sem).wait()
                pltpu.sync_copy(rows_v, acc_sh.at[idxd_v], add=True)
                return carry

            lax.fori_loop(0, nchunk, body, 0)
            plsc.subcore_barrier()
            pltpu.sync_copy(acc_sh.at[pl.ds(s * orows, orows)],
                            out_hbm.at[pl.ds(s * orows, orows)])

        @pl.when(c == 0)
        def _():
            run(h0_hbm, agg0_hbm)

        @pl.when(c == 1)
        def _():
            run(h1_hbm, agg1_hbm)

    return k


def _segment_sum_pool(h, src, dst, S, Epad):
    rows, D = h.shape
    Dh = D // 2
    zeros = jnp.zeros((_acc_rows(S) // _NTILE, Dh), jnp.float32)
    a0, a1 = _seg_sum_pool_kernel(S, Dh, Epad)(
        src, dst, h[:, :Dh], h[:, Dh:], zeros)
    return jnp.concatenate([a0, a1], axis=1)


@functools.cache
def _expand_kernel():
    mesh = plsc.VectorSubcoreMesh(core_axis_name="c", subcore_axis_name="s")
    bt = _B // 32
    nn = _NMAX * _NMAX
    nv = nn // 16

    @functools.partial(
        pl.kernel, mesh=mesh,
        out_type=jax.ShapeDtypeStruct((_B * nn,), jnp.float32),
        scratch_types=[
            pltpu.VMEM((nn,), jnp.int32),
            pltpu.VMEM((_LG,), jnp.float32),
            pltpu.VMEM((nn,), jnp.float32),
        ],
        compiler_params=pltpu.CompilerParams(needs_layout_passes=False),
    )
    def k(th_hbm, map_hbm, out_hbm, map_v, trow_v, orow_v):
        c = lax.axis_index("c")
        s = lax.axis_index("s")
        wid = s * 2 + c
        pltpu.sync_copy(map_hbm, map_v)

        def bbody(bi, carry):
            b = wid * bt + bi
            pltpu.sync_copy(th_hbm.at[pl.ds(b * _LG, _LG)], trow_v)

            def gbody(j, carry2):
                idx = map_v[pl.ds(j * 16, 16)]
                orow_v[pl.ds(j * 16, 16)] = plsc.load_gather(trow_v, [idx])
                return carry2

            lax.fori_loop(0, nv, gbody, 0)
            pltpu.sync_copy(orow_v, out_hbm.at[pl.ds(b * nn, nn)])
            return carry

        lax.fori_loop(0, bt, bbody, 0)

    return k


_BS = 1000


def _layer_a(h, aggs, W1, b1):
    n, din = h.shape
    dout = W1.shape[1]
    naggs = len(aggs)

    def body(h_ref, *refs):
        agg_refs = refs[:naggs]
        w_ref, b_ref, a_ref = refs[naggs:]
        if naggs == 2:
            agg = agg_refs[0][...] + agg_refs[1][...]
        else:
            agg = agg_refs[0][...]
        z = jnp.dot(h_ref[...] + agg, w_ref[...],
                    preferred_element_type=jnp.float32) + b_ref[...]
        a_ref[...] = jnp.where(z > 0, z, 0.2 * z)

    return pl.pallas_call(
        body,
        grid=(n // _BS,),
        in_specs=[pl.BlockSpec((_BS, din), lambda i: (i, 0))] * (1 + naggs)
                 + [pl.BlockSpec((din, dout), lambda i: (0, 0)),
                    pl.BlockSpec((1, dout), lambda i: (0, 0))],
        out_specs=pl.BlockSpec((_BS, dout), lambda i: (i, 0)),
        out_shape=jax.ShapeDtypeStruct((n, dout), jnp.float32),
    )(h, *aggs, W1, b1.reshape(1, -1))


def _layer_b(a, m, v, g, bt, W2, b2):
    n, din = a.shape
    dout = W2.shape[1]

    def body(a_ref, m_ref, v_ref, g_ref, t_ref, w_ref, b_ref, o_ref):
        xh = (a_ref[...] - m_ref[...]) / jnp.sqrt(v_ref[...] + _EPS) \
            * g_ref[...] + t_ref[...]
        z = jnp.dot(xh, w_ref[...],
                    preferred_element_type=jnp.float32) + b_ref[...]
        o_ref[...] = jnp.where(z > 0, z, 0.2 * z)

    return pl.pallas_call(
        body,
        grid=(n // _BS,),
        in_specs=[pl.BlockSpec((_BS, din), lambda i: (i, 0)),
                  pl.BlockSpec((1, din), lambda i: (0, 0)),
                  pl.BlockSpec((1, din), lambda i: (0, 0)),
                  pl.BlockSpec((1, din), lambda i: (0, 0)),
                  pl.BlockSpec((1, din), lambda i: (0, 0)),
                  pl.BlockSpec((din, dout), lambda i: (0, 0)),
                  pl.BlockSpec((1, dout), lambda i: (0, 0))],
        out_specs=pl.BlockSpec((_BS, dout), lambda i: (i, 0)),
        out_shape=jax.ShapeDtypeStruct((n, dout), jnp.float32),
    )(a, m.reshape(1, -1), v.reshape(1, -1), g.reshape(1, -1),
      bt.reshape(1, -1), W2, b2.reshape(1, -1))


def _decoder(pooled, bn_g, bn_b, fcW, fcb, dW0, db0, dW1, db1, dW2p, db2p):
    def body(p_ref, g_ref, b_ref, fw_ref, fb_ref, w0_ref, b0_ref,
             w1_ref, b1_ref, w2_ref, b2_ref, o_ref):
        p = p_ref[...]
        m = jnp.mean(p, 0, keepdims=True)
        v = jnp.mean((p - m) ** 2, 0, keepdims=True)
        ph = (p - m) / jnp.sqrt(v + _EPS) * g_ref[...] + b_ref[...]
        z = jnp.dot(ph, fw_ref[...],
                    preferred_element_type=jnp.float32) + fb_ref[...]
        d0 = jnp.maximum(jnp.dot(z, w0_ref[...],
                                 preferred_element_type=jnp.float32)
                         + b0_ref[...], 0.0)
        d1 = jnp.maximum(jnp.dot(d0, w1_ref[...],
                                 preferred_element_type=jnp.float32)
                         + b1_ref[...], 0.0)
        lg = jnp.dot(d1, w2_ref[...],
                     preferred_element_type=jnp.float32) + b2_ref[...]
        o_ref[...] = (lg > 0).astype(jnp.float32)

    return pl.pallas_call(
        body,
        out_shape=jax.ShapeDtypeStruct((_B, _LG), jnp.float32),
    )(pooled, bn_g.reshape(1, -1), bn_b.reshape(1, -1), fcW,
      fcb.reshape(1, -1), dW0, db0.reshape(1, -1), dW1, db1.reshape(1, -1),
      dW2p, db2p.reshape(1, -1))


def kernel(x, edge_index, batch, params):
    src = edge_index[0]
    dst = edge_index[1]
    e128 = _prep_edges(src, dst, _BOUNDS, _WID128, _WSF128)
    e256 = _prep_edges(src, dst, _BOUNDS256, _WID256, _WSF256)

    h = x
    for cp in params['convs']:
        if h.shape[1] == _D:
            aggs = _seg_sum_conv(h, *e128)
        else:
            aggs = _seg_sum_conv(h, *e256)
        a = _layer_a(h, list(aggs), cp['W1'], cp['b1'])
        m = jnp.mean(a, axis=0)
        v = jnp.var(a, axis=0)
        h = _layer_b(a, m, v, cp['g'], cp['bt'], cp['W2'], cp['b2'])

    psrc = jnp.concatenate(
        [jnp.arange(_N, dtype=jnp.int32),
         jnp.zeros((_NPAD - _N,), jnp.int32)])
    pdst = jnp.concatenate(
        [batch, jnp.full((_NPAD - _N,), _B, jnp.int32)])
    pooled = _segment_sum_pool(h, psrc, pdst, _B, _NPAD)

    dW2p = jnp.concatenate(
        [params['dW2'], jnp.zeros((_H, _LG - _NE), jnp.float32)], axis=1)
    db2p = jnp.concatenate(
        [params['db2'], jnp.full((_LG - _NE,), -1.0, jnp.float32)])
    th = _decoder(pooled, params['bn_g'], params['bn_b'],
                  params['fc_W'], params['fc_b'],
                  params['dW0'], params['db0'],
                  params['dW1'], params['db1'], dW2p, db2p)

    adj = _expand_kernel()(th.reshape(-1), jnp.asarray(_TRIU_MAP))
    return adj.reshape(_B, _NMAX, _NMAX)
